# Initial kernel scaffold; baseline (speedup 1.0000x reference)
import jax
import jax.numpy as jnp
from jax.experimental import pallas as pl


def kernel(x, edge_index, edge_attr, W1, b1, W2, b2, W_root, b_root):
    raise NotImplementedError("write your pallas kernel here")


# speedup vs baseline: 1.0615x; 1.0615x over previous
import functools

import jax
import jax.numpy as jnp
from jax import lax
from jax.experimental import pallas as pl
from jax.experimental.pallas import tpu as pltpu
from jax.experimental.pallas import tpu_sc as plsc

N_NODES = 10000
N_EDGES = 160000
DIM_IN = 32
DIM_OUT = 32
DIM_HID = 64
BOND_FDIM = 16

NC = 2
NS = 16
NW = NC * NS
EDGES_PER_W = N_EDGES // NW
CHUNK = 128
N_FULL = EDGES_PER_W // CHUNK
TAIL = EDGES_PER_W - N_FULL * CHUNK
ROWS_PER_TILE = N_NODES // NS

_SC_MESH = dict(core_axis_name="c", subcore_axis_name="s")


def _gather_body(x_hbm, src_hbm, out_hbm, idx_v, idx_t, rows_v, rows_t, sem):
    c = lax.axis_index("c")
    s = lax.axis_index("s")
    wid = s * NC + c
    base = wid * EDGES_PER_W

    def do_chunk(off, idx_ref, row_ref, size):
        pltpu.sync_copy(src_hbm.at[pl.ds(off, size)], idx_ref)
        pltpu.async_copy(x_hbm.at[idx_ref], row_ref, sem).wait()
        pltpu.sync_copy(row_ref, out_hbm.at[pl.ds(off, size)])

    def body(i, carry):
        do_chunk(base + i * CHUNK, idx_v, rows_v, CHUNK)
        return carry

    lax.fori_loop(0, N_FULL, body, 0)
    do_chunk(base + N_FULL * CHUNK, idx_t, rows_t, TAIL)


def _sc_gather(x, src):
    kern = pl.kernel(
        _gather_body,
        out_type=jax.ShapeDtypeStruct((N_EDGES, DIM_IN), jnp.float32),
        mesh=plsc.VectorSubcoreMesh(**_SC_MESH),
        scratch_types=[
            pltpu.VMEM((CHUNK,), jnp.int32),
            pltpu.VMEM((TAIL,), jnp.int32),
            pltpu.VMEM((CHUNK, DIM_IN), jnp.float32),
            pltpu.VMEM((TAIL, DIM_IN), jnp.float32),
            pltpu.SemaphoreType.DMA,
        ],
        compiler_params=pltpu.CompilerParams(use_tc_tiling_on_sc=False),
    )
    return kern(x, src)


LANES = 128
N_CHUNKS = N_EDGES // CHUNK
CHUNKS_PER_W = -(-N_CHUNKS // NW)
INIT_ROWS = (N_NODES // NS) // 8 * 8
INIT_REM = N_NODES - INIT_ROWS * NS


def _scatter_body(msg_hbm, dst_hbm, zeros_hbm, out_hbm, idx_v, rows_v, acc):
    c = lax.axis_index("c")
    s = lax.axis_index("s")
    wid = s * NC + c
    row0 = s * INIT_ROWS

    pltpu.sync_copy(zeros_hbm.at[pl.ds(0, INIT_ROWS)], acc.at[pl.ds(row0, INIT_ROWS)])

    @pl.when(s == 0)
    def _():
        pltpu.sync_copy(zeros_hbm.at[pl.ds(0, INIT_REM)],
                        acc.at[pl.ds(NS * INIT_ROWS, INIT_REM)])

    plsc.subcore_barrier()

    def body(k, carry):
        chunk_id = wid + k * NW

        @pl.when(chunk_id < N_CHUNKS)
        def _():
            off = chunk_id * CHUNK
            pltpu.sync_copy(dst_hbm.at[pl.ds(off, CHUNK)], idx_v)
            pltpu.sync_copy(msg_hbm.at[pl.ds(off, CHUNK)], rows_v)
            pltpu.sync_copy(rows_v, acc.at[idx_v], add=True)

        return carry

    lax.fori_loop(0, CHUNKS_PER_W, body, 0)
    plsc.subcore_barrier()

    pltpu.sync_copy(acc.at[pl.ds(row0, INIT_ROWS)],
                    out_hbm.at[pl.ds(c * N_NODES + row0, INIT_ROWS)])

    @pl.when(s == 0)
    def _():
        pltpu.sync_copy(acc.at[pl.ds(NS * INIT_ROWS, INIT_REM)],
                        out_hbm.at[pl.ds(c * N_NODES + NS * INIT_ROWS, INIT_REM)])


def _sc_scatter(msg, dst):
    zeros = jnp.zeros((INIT_ROWS, LANES), jnp.float32)
    kern = pl.kernel(
        _scatter_body,
        out_type=jax.ShapeDtypeStruct((NC * N_NODES, LANES), jnp.float32),
        mesh=plsc.VectorSubcoreMesh(**_SC_MESH),
        scratch_types=[
            pltpu.VMEM((CHUNK,), jnp.int32),
            pltpu.VMEM((CHUNK, LANES), jnp.float32),
            pltpu.VMEM_SHARED((N_NODES, LANES), jnp.float32),
        ],
    )
    return kern(msg, dst, zeros)


TE = 1000


def _msg_body(ea_ref, xj_ref, w1_ref, b1_ref, w2_ref, b2_ref, o_ref):
    h = jnp.dot(ea_ref[...], w1_ref[...], preferred_element_type=jnp.float32)
    h = jnp.maximum(h + b1_ref[...], 0.0)
    u = jnp.dot(h, w2_ref[...], preferred_element_type=jnp.float32)
    u = u + b2_ref[...]
    xb = lax.broadcast_in_dim(xj_ref[...], (TE, DIM_IN, DIM_OUT), (0, 1))
    p = u * jnp.reshape(xb, (TE, DIM_IN * DIM_OUT))
    w = (DIM_IN * DIM_OUT) // 2
    while w >= DIM_OUT:
        p = p[:, :w] + p[:, w:]
        w //= 2
    o_ref[...] = jnp.concatenate(
        [p, jnp.zeros((TE, LANES - DIM_OUT), jnp.float32)], axis=1)


def _tc_messages(ea, x_j, W1, b1, W2, b2):
    grid = (N_EDGES // TE,)
    return pl.pallas_call(
        _msg_body,
        grid=grid,
        in_specs=[
            pl.BlockSpec((TE, BOND_FDIM), lambda i: (i, 0)),
            pl.BlockSpec((TE, DIM_IN), lambda i: (i, 0)),
            pl.BlockSpec((BOND_FDIM, DIM_HID), lambda i: (0, 0)),
            pl.BlockSpec((1, DIM_HID), lambda i: (0, 0)),
            pl.BlockSpec((DIM_HID, DIM_IN * DIM_OUT), lambda i: (0, 0)),
            pl.BlockSpec((1, DIM_IN * DIM_OUT), lambda i: (0, 0)),
        ],
        out_specs=pl.BlockSpec((TE, LANES), lambda i: (i, 0)),
        out_shape=jax.ShapeDtypeStruct((N_EDGES, LANES), jnp.float32),
    )(ea, x_j, W1, b1.reshape(1, -1), W2, b2.reshape(1, -1))


TN = 1000


def _fin_body(a0_ref, a1_ref, x_ref, wr_ref, br_ref, o_ref):
    r = jnp.dot(x_ref[...], wr_ref[...], preferred_element_type=jnp.float32)
    agg = a0_ref[:, :DIM_OUT] + a1_ref[:, :DIM_OUT]
    o_ref[...] = jnp.maximum(agg + r + br_ref[...], 0.0)


def _tc_finalize(a0, a1, x, W_root, b_root):
    grid = (N_NODES // TN,)
    return pl.pallas_call(
        _fin_body,
        grid=grid,
        in_specs=[
            pl.BlockSpec((TN, LANES), lambda i: (i, 0)),
            pl.BlockSpec((TN, LANES), lambda i: (i, 0)),
            pl.BlockSpec((TN, DIM_IN), lambda i: (i, 0)),
            pl.BlockSpec((DIM_IN, DIM_OUT), lambda i: (0, 0)),
            pl.BlockSpec((1, DIM_OUT), lambda i: (0, 0)),
        ],
        out_specs=pl.BlockSpec((TN, DIM_OUT), lambda i: (i, 0)),
        out_shape=jax.ShapeDtypeStruct((N_NODES, DIM_OUT), jnp.float32),
    )(a0, a1, x, W_root, b_root.reshape(1, -1))


def kernel(x, edge_index, edge_attr, W1, b1, W2, b2, W_root, b_root):
    src = edge_index[0].astype(jnp.int32)
    dst = edge_index[1].astype(jnp.int32)
    x_j = _sc_gather(x, src)
    msg = _tc_messages(edge_attr, x_j, W1, b1, W2, b2)
    parts = _sc_scatter(msg, dst)
    return _tc_finalize(parts[:N_NODES], parts[N_NODES:], x, W_root, b_root)


# speedup vs baseline: 3.1047x; 2.9250x over previous
import functools

import numpy as np
import jax
import jax.numpy as jnp
from jax import lax
from jax.experimental import pallas as pl
from jax.experimental.pallas import tpu as pltpu
from jax.experimental.pallas import tpu_sc as plsc

N_NODES = 10000
N_EDGES = 160000
DIM_IN = 32
DIM_OUT = 32
DIM_HID = 64
BOND_FDIM = 16

NC = 2
NS = 16
NW = NC * NS
EDGES_PER_W = N_EDGES // NW
CHUNK = 128
N_FULL = EDGES_PER_W // CHUNK
TAIL = EDGES_PER_W - N_FULL * CHUNK
ROWS_PER_TILE = N_NODES // NS

_SC_MESH = dict(core_axis_name="c", subcore_axis_name="s")


def _gather_body(x_hbm, src_hbm, out_hbm, idx_v, idx_t, rows_v, rows_t, sem):
    c = lax.axis_index("c")
    s = lax.axis_index("s")
    wid = s * NC + c
    base = wid * EDGES_PER_W

    def do_chunk(off, idx_ref, row_ref, size):
        pltpu.sync_copy(src_hbm.at[pl.ds(off, size)], idx_ref)
        pltpu.async_copy(x_hbm.at[idx_ref], row_ref, sem).wait()
        pltpu.sync_copy(row_ref, out_hbm.at[pl.ds(off, size)])

    def body(i, carry):
        do_chunk(base + i * CHUNK, idx_v, rows_v, CHUNK)
        return carry

    lax.fori_loop(0, N_FULL, body, 0)
    do_chunk(base + N_FULL * CHUNK, idx_t, rows_t, TAIL)


def _sc_gather(x, src):
    kern = pl.kernel(
        _gather_body,
        out_type=jax.ShapeDtypeStruct((N_EDGES, DIM_IN), jnp.float32),
        mesh=plsc.VectorSubcoreMesh(**_SC_MESH),
        scratch_types=[
            pltpu.VMEM((CHUNK,), jnp.int32),
            pltpu.VMEM((TAIL,), jnp.int32),
            pltpu.VMEM((CHUNK, DIM_IN), jnp.float32),
            pltpu.VMEM((TAIL, DIM_IN), jnp.float32),
            pltpu.SemaphoreType.DMA,
        ],
        compiler_params=pltpu.CompilerParams(use_tc_tiling_on_sc=False),
    )
    return kern(x, src)


LANES = 128
N_CHUNKS = N_EDGES // CHUNK
CHUNKS_PER_W = -(-N_CHUNKS // NW)
INIT_ROWS = (N_NODES // NS) // 8 * 8
INIT_REM = N_NODES - INIT_ROWS * NS


def _scatter_body(msg_hbm, dst_hbm, zeros_hbm, out_hbm, idx_v, rows_v, acc):
    c = lax.axis_index("c")
    s = lax.axis_index("s")
    wid = s * NC + c
    row0 = s * INIT_ROWS

    pltpu.sync_copy(zeros_hbm.at[pl.ds(0, INIT_ROWS)], acc.at[pl.ds(row0, INIT_ROWS)])

    @pl.when(s == 0)
    def _():
        pltpu.sync_copy(zeros_hbm.at[pl.ds(0, INIT_REM)],
                        acc.at[pl.ds(NS * INIT_ROWS, INIT_REM)])

    plsc.subcore_barrier()

    def body(k, carry):
        chunk_id = wid + k * NW

        @pl.when(chunk_id < N_CHUNKS)
        def _():
            off = chunk_id * CHUNK
            pltpu.sync_copy(dst_hbm.at[pl.ds(off, CHUNK)], idx_v)
            pltpu.sync_copy(msg_hbm.at[pl.ds(off, CHUNK)], rows_v)
            pltpu.sync_copy(rows_v, acc.at[idx_v], add=True)

        return carry

    lax.fori_loop(0, CHUNKS_PER_W, body, 0)
    plsc.subcore_barrier()

    pltpu.sync_copy(acc.at[pl.ds(row0, INIT_ROWS)],
                    out_hbm.at[pl.ds(c * N_NODES + row0, INIT_ROWS)])

    @pl.when(s == 0)
    def _():
        pltpu.sync_copy(acc.at[pl.ds(NS * INIT_ROWS, INIT_REM)],
                        out_hbm.at[pl.ds(c * N_NODES + NS * INIT_ROWS, INIT_REM)])


def _sc_scatter(msg, dst):
    zeros = jnp.zeros((INIT_ROWS, LANES), jnp.float32)
    kern = pl.kernel(
        _scatter_body,
        out_type=jax.ShapeDtypeStruct((NC * N_NODES, LANES), jnp.float32),
        mesh=plsc.VectorSubcoreMesh(**_SC_MESH),
        scratch_types=[
            pltpu.VMEM((CHUNK,), jnp.int32),
            pltpu.VMEM((CHUNK, LANES), jnp.float32),
            pltpu.VMEM_SHARED((N_NODES, LANES), jnp.float32),
        ],
    )
    return kern(msg, dst, zeros)


TE = 1000


def _msg_body(ea_ref, xj_ref, w1_ref, b1_ref, w2_ref, b2_ref, rep_ref, o_ref):
    h = jnp.dot(ea_ref[...], w1_ref[...], preferred_element_type=jnp.float32)
    h = jnp.maximum(h + b1_ref[...], 0.0)
    u = jnp.dot(h, w2_ref[...], preferred_element_type=jnp.float32)
    u = u + b2_ref[...]
    xb = jnp.dot(xj_ref[...], rep_ref[...], preferred_element_type=jnp.float32)
    p = u * xb
    w = (DIM_IN * DIM_OUT) // 2
    while w >= DIM_OUT:
        p = p[:, :w] + p[:, w:]
        w //= 2
    o_ref[...] = jnp.concatenate(
        [p, jnp.zeros((TE, LANES - DIM_OUT), jnp.float32)], axis=1)


def _tc_messages(ea, x_j, W1, b1, W2, b2):
    rep = jnp.asarray(
        np.kron(np.eye(DIM_IN, dtype=np.float32),
                np.ones((1, DIM_OUT), dtype=np.float32)))
    grid = (N_EDGES // TE,)
    return pl.pallas_call(
        _msg_body,
        grid=grid,
        in_specs=[
            pl.BlockSpec((TE, BOND_FDIM), lambda i: (i, 0)),
            pl.BlockSpec((TE, DIM_IN), lambda i: (i, 0)),
            pl.BlockSpec((BOND_FDIM, DIM_HID), lambda i: (0, 0)),
            pl.BlockSpec((1, DIM_HID), lambda i: (0, 0)),
            pl.BlockSpec((DIM_HID, DIM_IN * DIM_OUT), lambda i: (0, 0)),
            pl.BlockSpec((1, DIM_IN * DIM_OUT), lambda i: (0, 0)),
            pl.BlockSpec((DIM_IN, DIM_IN * DIM_OUT), lambda i: (0, 0)),
        ],
        out_specs=pl.BlockSpec((TE, LANES), lambda i: (i, 0)),
        out_shape=jax.ShapeDtypeStruct((N_EDGES, LANES), jnp.float32),
    )(ea, x_j, W1, b1.reshape(1, -1), W2, b2.reshape(1, -1), rep)


TN = 1000


def _fin_body(a0_ref, a1_ref, x_ref, wr_ref, br_ref, o_ref):
    r = jnp.dot(x_ref[...], wr_ref[...], preferred_element_type=jnp.float32)
    agg = a0_ref[:, :DIM_OUT] + a1_ref[:, :DIM_OUT]
    o_ref[...] = jnp.maximum(agg + r + br_ref[...], 0.0)


def _tc_finalize(a0, a1, x, W_root, b_root):
    grid = (N_NODES // TN,)
    return pl.pallas_call(
        _fin_body,
        grid=grid,
        in_specs=[
            pl.BlockSpec((TN, LANES), lambda i: (i, 0)),
            pl.BlockSpec((TN, LANES), lambda i: (i, 0)),
            pl.BlockSpec((TN, DIM_IN), lambda i: (i, 0)),
            pl.BlockSpec((DIM_IN, DIM_OUT), lambda i: (0, 0)),
            pl.BlockSpec((1, DIM_OUT), lambda i: (0, 0)),
        ],
        out_specs=pl.BlockSpec((TN, DIM_OUT), lambda i: (i, 0)),
        out_shape=jax.ShapeDtypeStruct((N_NODES, DIM_OUT), jnp.float32),
    )(a0, a1, x, W_root, b_root.reshape(1, -1))


def kernel(x, edge_index, edge_attr, W1, b1, W2, b2, W_root, b_root):
    src = edge_index[0].astype(jnp.int32)
    dst = edge_index[1].astype(jnp.int32)
    x_j = _sc_gather(x, src)
    msg = _tc_messages(edge_attr, x_j, W1, b1, W2, b2)
    parts = _sc_scatter(msg, dst)
    return _tc_finalize(parts[:N_NODES], parts[N_NODES:], x, W_root, b_root)


# speedup vs baseline: 3.1305x; 1.0083x over previous
import functools

import numpy as np
import jax
import jax.numpy as jnp
from jax import lax
from jax.experimental import pallas as pl
from jax.experimental.pallas import tpu as pltpu
from jax.experimental.pallas import tpu_sc as plsc

N_NODES = 10000
N_EDGES = 160000
DIM_IN = 32
DIM_OUT = 32
DIM_HID = 64
BOND_FDIM = 16

NC = 2
NS = 16
NW = NC * NS
CHUNK = 128
LANES = 128
N_CHUNKS = N_EDGES // CHUNK
CHUNKS_PER_W = -(-N_CHUNKS // NW)

_SC_MESH = dict(core_axis_name="c", subcore_axis_name="s")


def _gather_body(x_hbm, src_hbm, out_hbm, idx_v, rows_v, sem):
    c = lax.axis_index("c")
    s = lax.axis_index("s")
    wid = s * NC + c

    def body(k, carry):
        chunk_id = wid + k * NW

        @pl.when(chunk_id < N_CHUNKS)
        def _():
            off = chunk_id * CHUNK
            pltpu.sync_copy(src_hbm.at[pl.ds(off, CHUNK)], idx_v)
            pltpu.async_copy(x_hbm.at[idx_v], rows_v, sem).wait()
            pltpu.sync_copy(rows_v, out_hbm.at[pl.ds(off, CHUNK)])

        return carry

    lax.fori_loop(0, CHUNKS_PER_W, body, 0)


def _sc_gather(x128, src):
    kern = pl.kernel(
        _gather_body,
        out_type=jax.ShapeDtypeStruct((N_EDGES, LANES), jnp.float32),
        mesh=plsc.VectorSubcoreMesh(**_SC_MESH),
        scratch_types=[
            pltpu.VMEM((CHUNK,), jnp.int32),
            pltpu.VMEM((CHUNK, LANES), jnp.float32),
            pltpu.SemaphoreType.DMA,
        ],
    )
    return kern(x128, src)


INIT_ROWS = (N_NODES // NS) // 8 * 8
INIT_REM = N_NODES - INIT_ROWS * NS


def _scatter_body(msg_hbm, dst_hbm, zeros_hbm, out_hbm, idx_v, rows_v, acc):
    c = lax.axis_index("c")
    s = lax.axis_index("s")
    wid = s * NC + c
    row0 = s * INIT_ROWS

    pltpu.sync_copy(zeros_hbm.at[pl.ds(0, INIT_ROWS)], acc.at[pl.ds(row0, INIT_ROWS)])

    @pl.when(s == 0)
    def _():
        pltpu.sync_copy(zeros_hbm.at[pl.ds(0, INIT_REM)],
                        acc.at[pl.ds(NS * INIT_ROWS, INIT_REM)])

    plsc.subcore_barrier()

    def body(k, carry):
        chunk_id = wid + k * NW

        @pl.when(chunk_id < N_CHUNKS)
        def _():
            off = chunk_id * CHUNK
            pltpu.sync_copy(dst_hbm.at[pl.ds(off, CHUNK)], idx_v)
            pltpu.sync_copy(msg_hbm.at[pl.ds(off, CHUNK)], rows_v)
            pltpu.sync_copy(rows_v, acc.at[idx_v], add=True)

        return carry

    lax.fori_loop(0, CHUNKS_PER_W, body, 0)
    plsc.subcore_barrier()

    pltpu.sync_copy(acc.at[pl.ds(row0, INIT_ROWS)],
                    out_hbm.at[pl.ds(c * N_NODES + row0, INIT_ROWS)])

    @pl.when(s == 0)
    def _():
        pltpu.sync_copy(acc.at[pl.ds(NS * INIT_ROWS, INIT_REM)],
                        out_hbm.at[pl.ds(c * N_NODES + NS * INIT_ROWS, INIT_REM)])


def _sc_scatter(msg, dst):
    zeros = jnp.zeros((INIT_ROWS, LANES), jnp.float32)
    kern = pl.kernel(
        _scatter_body,
        out_type=jax.ShapeDtypeStruct((NC * N_NODES, LANES), jnp.float32),
        mesh=plsc.VectorSubcoreMesh(**_SC_MESH),
        scratch_types=[
            pltpu.VMEM((CHUNK,), jnp.int32),
            pltpu.VMEM((CHUNK, LANES), jnp.float32),
            pltpu.VMEM_SHARED((N_NODES, LANES), jnp.float32),
        ],
    )
    return kern(msg, dst, zeros)


TE = 1000


def _msg_body(ea_ref, xj_ref, w1_ref, b1_ref, w2_ref, b2_ref, rep_ref, o_ref):
    h = jnp.dot(ea_ref[...], w1_ref[...], preferred_element_type=jnp.float32)
    h = jnp.maximum(h + b1_ref[...], 0.0)
    u = jnp.dot(h.astype(jnp.bfloat16), w2_ref[...],
                preferred_element_type=jnp.float32)
    u = u + b2_ref[...]
    xj = xj_ref[:, :DIM_IN]
    xb = jnp.dot(xj, rep_ref[...], preferred_element_type=jnp.float32)
    p = u * xb
    w = (DIM_IN * DIM_OUT) // 2
    while w >= DIM_OUT:
        p = p[:, :w] + p[:, w:]
        w //= 2
    o_ref[...] = jnp.concatenate(
        [p, jnp.zeros((TE, LANES - DIM_OUT), jnp.float32)], axis=1)


def _tc_messages(ea, x_j, W1, b1, W2, b2):
    rep = jnp.asarray(
        np.kron(np.eye(DIM_IN, dtype=np.float32),
                np.ones((1, DIM_OUT), dtype=np.float32)))
    grid = (N_EDGES // TE,)
    return pl.pallas_call(
        _msg_body,
        grid=grid,
        in_specs=[
            pl.BlockSpec((TE, BOND_FDIM), lambda i: (i, 0)),
            pl.BlockSpec((TE, LANES), lambda i: (i, 0)),
            pl.BlockSpec((BOND_FDIM, DIM_HID), lambda i: (0, 0)),
            pl.BlockSpec((1, DIM_HID), lambda i: (0, 0)),
            pl.BlockSpec((DIM_HID, DIM_IN * DIM_OUT), lambda i: (0, 0)),
            pl.BlockSpec((1, DIM_IN * DIM_OUT), lambda i: (0, 0)),
            pl.BlockSpec((DIM_IN, DIM_IN * DIM_OUT), lambda i: (0, 0)),
        ],
        out_specs=pl.BlockSpec((TE, LANES), lambda i: (i, 0)),
        out_shape=jax.ShapeDtypeStruct((N_EDGES, LANES), jnp.float32),
    )(ea, x_j, W1, b1.reshape(1, -1), W2.astype(jnp.bfloat16),
      b2.reshape(1, -1), rep)


TN = 1000


def _fin_body(a0_ref, a1_ref, x_ref, wr_ref, br_ref, o_ref):
    r = jnp.dot(x_ref[...], wr_ref[...], preferred_element_type=jnp.float32)
    agg = a0_ref[:, :DIM_OUT] + a1_ref[:, :DIM_OUT]
    o_ref[...] = jnp.maximum(agg + r + br_ref[...], 0.0)


def _tc_finalize(a0, a1, x, W_root, b_root):
    grid = (N_NODES // TN,)
    return pl.pallas_call(
        _fin_body,
        grid=grid,
        in_specs=[
            pl.BlockSpec((TN, LANES), lambda i: (i, 0)),
            pl.BlockSpec((TN, LANES), lambda i: (i, 0)),
            pl.BlockSpec((TN, DIM_IN), lambda i: (i, 0)),
            pl.BlockSpec((DIM_IN, DIM_OUT), lambda i: (0, 0)),
            pl.BlockSpec((1, DIM_OUT), lambda i: (0, 0)),
        ],
        out_specs=pl.BlockSpec((TN, DIM_OUT), lambda i: (i, 0)),
        out_shape=jax.ShapeDtypeStruct((N_NODES, DIM_OUT), jnp.float32),
    )(a0, a1, x, W_root, b_root.reshape(1, -1))


def kernel(x, edge_index, edge_attr, W1, b1, W2, b2, W_root, b_root):
    src = edge_index[0].astype(jnp.int32)
    dst = edge_index[1].astype(jnp.int32)
    x128 = jnp.concatenate(
        [x, jnp.zeros((N_NODES, LANES - DIM_IN), jnp.float32)], axis=1)
    x_j = _sc_gather(x128, src)
    msg = _tc_messages(edge_attr, x_j, W1, b1, W2, b2)
    parts = _sc_scatter(msg, dst)
    return _tc_finalize(parts[:N_NODES], parts[N_NODES:], x, W_root, b_root)


# speedup vs baseline: 3.9388x; 1.2582x over previous
import functools

import numpy as np
import jax
import jax.numpy as jnp
from jax import lax
from jax.experimental import pallas as pl
from jax.experimental.pallas import tpu as pltpu
from jax.experimental.pallas import tpu_sc as plsc

N_NODES = 10000
N_EDGES = 160000
DIM_IN = 32
DIM_OUT = 32
DIM_HID = 64
BOND_FDIM = 16

NC = 2
NS = 16
NW = NC * NS
CHUNK = 128
LANES = 128
N_CHUNKS = N_EDGES // CHUNK
CHUNKS_PER_W = -(-N_CHUNKS // NW)

_SC_MESH = dict(core_axis_name="c", subcore_axis_name="s")


STEADY = N_CHUNKS // NW
EXTRA_W = N_CHUNKS - STEADY * NW


def _load_all_idx(src_hbm, idx_all, wid, sem):
    def fire(k, carry):
        off = (wid + k * NW) * CHUNK
        pltpu.make_async_copy(src_hbm.at[pl.ds(off, CHUNK)],
                              idx_all.at[k], sem).start()
        return carry

    lax.fori_loop(0, STEADY, fire, 0)

    def drain(k, carry):
        pltpu.make_async_copy(src_hbm.at[pl.ds(0, CHUNK)],
                              idx_all.at[k], sem).wait()
        return carry

    lax.fori_loop(0, STEADY, drain, 0)


def _gather_body(x_hbm, src_hbm, out_hbm, idx_all, rows0, rows1, sem,
                 g0, g1):
    c = lax.axis_index("c")
    s = lax.axis_index("s")
    wid = s * NC + c
    _load_all_idx(src_hbm, idx_all, wid, sem)
    rows = (rows0, rows1)
    gsem = (g0, g1)

    def start(k, b):
        pltpu.make_async_copy(x_hbm.at[idx_all.at[k]], rows[b], gsem[b]).start()

    def finish(k, b):
        pltpu.make_async_copy(x_hbm.at[idx_all.at[k]], rows[b], gsem[b]).wait()
        off = (wid + k * NW) * CHUNK
        pltpu.sync_copy(rows[b], out_hbm.at[pl.ds(off, CHUNK)])

    start(0, 0)

    def body(kp, carry):
        k = 2 * kp
        start(k + 1, 1)
        finish(k, 0)
        start(k + 2, 0)
        finish(k + 1, 1)
        return carry

    lax.fori_loop(0, (STEADY - 1) // 2, body, 0)
    finish(STEADY - 1, 0)

    @pl.when(wid < EXTRA_W)
    def _():
        off = (wid + STEADY * NW) * CHUNK
        pltpu.sync_copy(src_hbm.at[pl.ds(off, CHUNK)], idx_all.at[0])
        pltpu.async_copy(x_hbm.at[idx_all.at[0]], rows0, g0).wait()
        pltpu.sync_copy(rows0, out_hbm.at[pl.ds(off, CHUNK)])


def _sc_gather(x128, src):
    kern = pl.kernel(
        _gather_body,
        out_type=jax.ShapeDtypeStruct((N_EDGES, LANES), jnp.float32),
        mesh=plsc.VectorSubcoreMesh(**_SC_MESH),
        scratch_types=[
            pltpu.VMEM((STEADY, CHUNK), jnp.int32),
            pltpu.VMEM((CHUNK, LANES), jnp.float32),
            pltpu.VMEM((CHUNK, LANES), jnp.float32),
            pltpu.SemaphoreType.DMA,
            pltpu.SemaphoreType.DMA,
            pltpu.SemaphoreType.DMA,
        ],
    )
    return kern(x128, src)


INIT_ROWS = (N_NODES // NS) // 8 * 8
INIT_REM = N_NODES - INIT_ROWS * NS


def _scatter_body(msg_hbm, dst_hbm, zeros_hbm, out_hbm, idx_all,
                  rows0, rows1, acc, sem, g0, g1):
    c = lax.axis_index("c")
    s = lax.axis_index("s")
    wid = s * NC + c
    row0 = s * INIT_ROWS

    pltpu.sync_copy(zeros_hbm.at[pl.ds(0, INIT_ROWS)], acc.at[pl.ds(row0, INIT_ROWS)])

    @pl.when(s == 0)
    def _():
        pltpu.sync_copy(zeros_hbm.at[pl.ds(0, INIT_REM)],
                        acc.at[pl.ds(NS * INIT_ROWS, INIT_REM)])

    _load_all_idx(dst_hbm, idx_all, wid, sem)
    plsc.subcore_barrier()
    rows = (rows0, rows1)
    gsem = (g0, g1)

    def start(k, b):
        off = (wid + k * NW) * CHUNK
        pltpu.make_async_copy(msg_hbm.at[pl.ds(off, CHUNK)],
                              rows[b], gsem[b]).start()

    def finish(k, b):
        off = (wid + k * NW) * CHUNK
        pltpu.make_async_copy(msg_hbm.at[pl.ds(off, CHUNK)],
                              rows[b], gsem[b]).wait()
        pltpu.sync_copy(rows[b], acc.at[idx_all.at[k]], add=True)

    start(0, 0)

    def body(kp, carry):
        k = 2 * kp
        start(k + 1, 1)
        finish(k, 0)
        start(k + 2, 0)
        finish(k + 1, 1)
        return carry

    lax.fori_loop(0, (STEADY - 1) // 2, body, 0)
    finish(STEADY - 1, 0)

    @pl.when(wid < EXTRA_W)
    def _():
        off = (wid + STEADY * NW) * CHUNK
        pltpu.sync_copy(dst_hbm.at[pl.ds(off, CHUNK)], idx_all.at[0])
        pltpu.sync_copy(msg_hbm.at[pl.ds(off, CHUNK)], rows0)
        pltpu.sync_copy(rows0, acc.at[idx_all.at[0]], add=True)

    plsc.subcore_barrier()

    pltpu.sync_copy(acc.at[pl.ds(row0, INIT_ROWS)],
                    out_hbm.at[pl.ds(c * N_NODES + row0, INIT_ROWS)])

    @pl.when(s == 0)
    def _():
        pltpu.sync_copy(acc.at[pl.ds(NS * INIT_ROWS, INIT_REM)],
                        out_hbm.at[pl.ds(c * N_NODES + NS * INIT_ROWS, INIT_REM)])


def _sc_scatter(msg, dst):
    zeros = jnp.zeros((INIT_ROWS, LANES), jnp.float32)
    kern = pl.kernel(
        _scatter_body,
        out_type=jax.ShapeDtypeStruct((NC * N_NODES, LANES), jnp.float32),
        mesh=plsc.VectorSubcoreMesh(**_SC_MESH),
        scratch_types=[
            pltpu.VMEM((STEADY, CHUNK), jnp.int32),
            pltpu.VMEM((CHUNK, LANES), jnp.float32),
            pltpu.VMEM((CHUNK, LANES), jnp.float32),
            pltpu.VMEM_SHARED((N_NODES, LANES), jnp.float32),
            pltpu.SemaphoreType.DMA,
            pltpu.SemaphoreType.DMA,
            pltpu.SemaphoreType.DMA,
        ],
    )
    return kern(msg, dst, zeros)


TE = 2000


def _msg_body(ea_ref, xj_ref, w1_ref, b1_ref, w2_ref, b2_ref, rep_ref, o_ref):
    h = jnp.dot(ea_ref[...], w1_ref[...], preferred_element_type=jnp.float32)
    h = jnp.maximum(h + b1_ref[...], 0.0)
    u = jnp.dot(h.astype(jnp.bfloat16), w2_ref[...],
                preferred_element_type=jnp.float32)
    u = u + b2_ref[...]
    xj = xj_ref[:, :DIM_IN]
    xb = jnp.dot(xj, rep_ref[...], preferred_element_type=jnp.float32)
    p = u * xb
    w = (DIM_IN * DIM_OUT) // 2
    while w >= DIM_OUT:
        p = p[:, :w] + p[:, w:]
        w //= 2
    o_ref[...] = jnp.concatenate(
        [p, jnp.zeros((TE, LANES - DIM_OUT), jnp.float32)], axis=1)


def _tc_messages(ea, x_j, W1, b1, W2, b2):
    rep = jnp.asarray(
        np.kron(np.eye(DIM_IN, dtype=np.float32),
                np.ones((1, DIM_OUT), dtype=np.float32)))
    grid = (N_EDGES // TE,)
    return pl.pallas_call(
        _msg_body,
        grid=grid,
        in_specs=[
            pl.BlockSpec((TE, BOND_FDIM), lambda i: (i, 0)),
            pl.BlockSpec((TE, LANES), lambda i: (i, 0)),
            pl.BlockSpec((BOND_FDIM, DIM_HID), lambda i: (0, 0)),
            pl.BlockSpec((1, DIM_HID), lambda i: (0, 0)),
            pl.BlockSpec((DIM_HID, DIM_IN * DIM_OUT), lambda i: (0, 0)),
            pl.BlockSpec((1, DIM_IN * DIM_OUT), lambda i: (0, 0)),
            pl.BlockSpec((DIM_IN, DIM_IN * DIM_OUT), lambda i: (0, 0)),
        ],
        out_specs=pl.BlockSpec((TE, LANES), lambda i: (i, 0)),
        out_shape=jax.ShapeDtypeStruct((N_EDGES, LANES), jnp.float32),
    )(ea, x_j, W1, b1.reshape(1, -1), W2.astype(jnp.bfloat16),
      b2.reshape(1, -1), rep)


TN = 1000


def _fin_body(a0_ref, a1_ref, x_ref, wr_ref, br_ref, o_ref):
    r = jnp.dot(x_ref[...], wr_ref[...], preferred_element_type=jnp.float32)
    agg = a0_ref[:, :DIM_OUT] + a1_ref[:, :DIM_OUT]
    o_ref[...] = jnp.maximum(agg + r + br_ref[...], 0.0)


def _tc_finalize(parts, x, W_root, b_root):
    grid = (N_NODES // TN,)
    return pl.pallas_call(
        _fin_body,
        grid=grid,
        in_specs=[
            pl.BlockSpec((TN, LANES), lambda i: (i, 0)),
            pl.BlockSpec((TN, LANES), lambda i: (i + N_NODES // TN, 0)),
            pl.BlockSpec((TN, DIM_IN), lambda i: (i, 0)),
            pl.BlockSpec((DIM_IN, DIM_OUT), lambda i: (0, 0)),
            pl.BlockSpec((1, DIM_OUT), lambda i: (0, 0)),
        ],
        out_specs=pl.BlockSpec((TN, DIM_OUT), lambda i: (i, 0)),
        out_shape=jax.ShapeDtypeStruct((N_NODES, DIM_OUT), jnp.float32),
    )(parts, parts, x, W_root, b_root.reshape(1, -1))


def kernel(x, edge_index, edge_attr, W1, b1, W2, b2, W_root, b_root):
    src = edge_index[0].astype(jnp.int32)
    dst = edge_index[1].astype(jnp.int32)
    x128 = jnp.concatenate(
        [x, jnp.zeros((N_NODES, LANES - DIM_IN), jnp.float32)], axis=1)
    x_j = _sc_gather(x128, src)
    msg = _tc_messages(edge_attr, x_j, W1, b1, W2, b2)
    parts = _sc_scatter(msg, dst)
    return _tc_finalize(parts, x, W_root, b_root)


# speedup vs baseline: 4.3508x; 1.1046x over previous
import numpy as np
import jax
import jax.numpy as jnp
from jax import lax
from jax.experimental import pallas as pl
from jax.experimental.pallas import tpu as pltpu
from jax.experimental.pallas import tpu_sc as plsc

N_NODES = 10000
N_EDGES = 160000
DIM_IN = 32
DIM_OUT = 32
DIM_HID = 64
BOND_FDIM = 16

NC = 2
NS = 16
NW = NC * NS
CHUNK = 128
PACK = 4
LANES = PACK * DIM_IN
N_CHUNKS = N_EDGES // CHUNK
STEADY = N_CHUNKS // NW
EXTRA_W = N_CHUNKS - STEADY * NW

_SC_MESH = dict(core_axis_name="c", subcore_axis_name="s")
_SC_PARAMS = pltpu.CompilerParams(use_tc_tiling_on_sc=False)


def _load_all_idx(src_hbm, idx_all, wid, sem):
    def fire(k, carry):
        off = (wid + k * NW) * CHUNK
        pltpu.make_async_copy(src_hbm.at[pl.ds(off, CHUNK)],
                              idx_all.at[k], sem).start()
        return carry

    lax.fori_loop(0, STEADY, fire, 0)

    def drain(k, carry):
        pltpu.make_async_copy(src_hbm.at[pl.ds(0, CHUNK)],
                              idx_all.at[k], sem).wait()
        return carry

    lax.fori_loop(0, STEADY, drain, 0)


def _gather_body(x_hbm, src_hbm, out_hbm, idx_all, rows0, rows1, sem,
                 g0, g1):
    c = lax.axis_index("c")
    s = lax.axis_index("s")
    wid = s * NC + c
    _load_all_idx(src_hbm, idx_all, wid, sem)
    rows = (rows0, rows1)
    gsem = (g0, g1)

    def start(k, b):
        pltpu.make_async_copy(x_hbm.at[idx_all.at[k]], rows[b], gsem[b]).start()

    def finish(k, b):
        pltpu.make_async_copy(x_hbm.at[idx_all.at[k]], rows[b], gsem[b]).wait()
        off = (wid + k * NW) * CHUNK
        pltpu.sync_copy(rows[b], out_hbm.at[pl.ds(off, CHUNK)])

    start(0, 0)

    def body(kp, carry):
        k = 2 * kp
        start(k + 1, 1)
        finish(k, 0)
        start(k + 2, 0)
        finish(k + 1, 1)
        return carry

    lax.fori_loop(0, (STEADY - 1) // 2, body, 0)
    finish(STEADY - 1, 0)

    @pl.when(wid < EXTRA_W)
    def _():
        off = (wid + STEADY * NW) * CHUNK
        pltpu.sync_copy(src_hbm.at[pl.ds(off, CHUNK)], idx_all.at[0])
        pltpu.async_copy(x_hbm.at[idx_all.at[0]], rows0, g0).wait()
        pltpu.sync_copy(rows0, out_hbm.at[pl.ds(off, CHUNK)])


def _sc_gather(x, src):
    kern = pl.kernel(
        _gather_body,
        out_type=jax.ShapeDtypeStruct((N_EDGES, DIM_IN), jnp.float32),
        mesh=plsc.VectorSubcoreMesh(**_SC_MESH),
        scratch_types=[
            pltpu.VMEM((STEADY, CHUNK), jnp.int32),
            pltpu.VMEM((CHUNK, DIM_IN), jnp.float32),
            pltpu.VMEM((CHUNK, DIM_IN), jnp.float32),
            pltpu.SemaphoreType.DMA,
            pltpu.SemaphoreType.DMA,
            pltpu.SemaphoreType.DMA,
        ],
        compiler_params=_SC_PARAMS,
    )
    return kern(x, src)


INIT_ROWS = (N_NODES // NS) // 8 * 8
INIT_REM = N_NODES - INIT_ROWS * NS


def _scatter_body(msg_hbm, dst_hbm, zeros_hbm, out_hbm, idx_all,
                  rows0, rows1, acc, sem, g0, g1):
    c = lax.axis_index("c")
    s = lax.axis_index("s")
    wid = s * NC + c
    row0 = s * INIT_ROWS

    pltpu.sync_copy(zeros_hbm.at[pl.ds(0, INIT_ROWS)], acc.at[pl.ds(row0, INIT_ROWS)])

    @pl.when(s == 0)
    def _():
        pltpu.sync_copy(zeros_hbm.at[pl.ds(0, INIT_REM)],
                        acc.at[pl.ds(NS * INIT_ROWS, INIT_REM)])

    _load_all_idx(dst_hbm, idx_all, wid, sem)
    plsc.subcore_barrier()
    rows = (rows0, rows1)
    gsem = (g0, g1)

    def start(k, b):
        off = (wid + k * NW) * CHUNK
        pltpu.make_async_copy(msg_hbm.at[pl.ds(off, CHUNK)],
                              rows[b], gsem[b]).start()

    def finish(k, b):
        off = (wid + k * NW) * CHUNK
        pltpu.make_async_copy(msg_hbm.at[pl.ds(off, CHUNK)],
                              rows[b], gsem[b]).wait()
        pltpu.sync_copy(rows[b], acc.at[idx_all.at[k]], add=True)

    start(0, 0)

    def body(kp, carry):
        k = 2 * kp
        start(k + 1, 1)
        finish(k, 0)
        start(k + 2, 0)
        finish(k + 1, 1)
        return carry

    lax.fori_loop(0, (STEADY - 1) // 2, body, 0)
    finish(STEADY - 1, 0)

    @pl.when(wid < EXTRA_W)
    def _():
        off = (wid + STEADY * NW) * CHUNK
        pltpu.sync_copy(dst_hbm.at[pl.ds(off, CHUNK)], idx_all.at[0])
        pltpu.sync_copy(msg_hbm.at[pl.ds(off, CHUNK)], rows0)
        pltpu.sync_copy(rows0, acc.at[idx_all.at[0]], add=True)

    plsc.subcore_barrier()

    pltpu.sync_copy(acc.at[pl.ds(row0, INIT_ROWS)],
                    out_hbm.at[pl.ds(c * N_NODES + row0, INIT_ROWS)])

    @pl.when(s == 0)
    def _():
        pltpu.sync_copy(acc.at[pl.ds(NS * INIT_ROWS, INIT_REM)],
                        out_hbm.at[pl.ds(c * N_NODES + NS * INIT_ROWS, INIT_REM)])


def _sc_scatter(msg, dst):
    zeros = jnp.zeros((INIT_ROWS, DIM_OUT), jnp.float32)
    kern = pl.kernel(
        _scatter_body,
        out_type=jax.ShapeDtypeStruct((NC * N_NODES, DIM_OUT), jnp.float32),
        mesh=plsc.VectorSubcoreMesh(**_SC_MESH),
        scratch_types=[
            pltpu.VMEM((STEADY, CHUNK), jnp.int32),
            pltpu.VMEM((CHUNK, DIM_OUT), jnp.float32),
            pltpu.VMEM((CHUNK, DIM_OUT), jnp.float32),
            pltpu.VMEM_SHARED((N_NODES, DIM_OUT), jnp.float32),
            pltpu.SemaphoreType.DMA,
            pltpu.SemaphoreType.DMA,
            pltpu.SemaphoreType.DMA,
        ],
        compiler_params=_SC_PARAMS,
    )
    return kern(msg, dst, zeros)


TE = 1600
TR = TE // PACK
N_ROWS = N_EDGES // PACK
HID4 = PACK * DIM_HID
UW = DIM_IN * DIM_OUT
UW4 = PACK * UW


def _msg_body(ea_ref, xj_ref, w1_ref, b1_ref, w2_ref, b2_ref, rep_ref, o_ref):
    h = jnp.dot(ea_ref[...], w1_ref[...], preferred_element_type=jnp.float32)
    h = jnp.maximum(h + b1_ref[...], 0.0)
    u = jnp.dot(h.astype(jnp.bfloat16), w2_ref[...],
                preferred_element_type=jnp.float32)
    u = u + b2_ref[...]
    xb = jnp.dot(xj_ref[...], rep_ref[...], preferred_element_type=jnp.float32)
    p = u * xb
    blocks = []
    for q in range(PACK):
        b = p[:, q * UW:(q + 1) * UW]
        w = UW // 2
        while w >= DIM_OUT:
            b = b[:, :w] + b[:, w:]
            w //= 2
        blocks.append(b)
    o_ref[...] = jnp.concatenate(blocks, axis=1)


def _tc_messages(ea4, xj4, W1, b1, W2, b2):
    eye = jnp.eye(PACK, dtype=jnp.float32)
    w1bd = jnp.kron(eye, W1)
    w2bd = jnp.kron(eye, W2).astype(jnp.bfloat16)
    b1_4 = jnp.tile(b1, PACK).reshape(1, HID4)
    b2_4 = jnp.tile(b2, PACK).reshape(1, UW4)
    rep = np.kron(np.eye(PACK, dtype=np.float32),
                  np.kron(np.eye(DIM_IN, dtype=np.float32),
                          np.ones((1, DIM_OUT), dtype=np.float32)))
    rep = jnp.asarray(rep)
    grid = (N_ROWS // TR,)
    return pl.pallas_call(
        _msg_body,
        grid=grid,
        in_specs=[
            pl.BlockSpec((TR, PACK * BOND_FDIM), lambda i: (i, 0)),
            pl.BlockSpec((TR, LANES), lambda i: (i, 0)),
            pl.BlockSpec((DIM_HID, HID4), lambda i: (0, 0)),
            pl.BlockSpec((1, HID4), lambda i: (0, 0)),
            pl.BlockSpec((HID4, UW4), lambda i: (0, 0)),
            pl.BlockSpec((1, UW4), lambda i: (0, 0)),
            pl.BlockSpec((LANES, UW4), lambda i: (0, 0)),
        ],
        out_specs=pl.BlockSpec((TR, LANES), lambda i: (i, 0)),
        out_shape=jax.ShapeDtypeStruct((N_ROWS, LANES), jnp.float32),
    )(ea4, xj4, w1bd, b1_4, w2bd, b2_4, rep)


TN = 1000


def _fin_body(a0_ref, a1_ref, x_ref, wr_ref, br_ref, o_ref):
    r = jnp.dot(x_ref[...], wr_ref[...], preferred_element_type=jnp.float32)
    agg = a0_ref[...] + a1_ref[...]
    o_ref[...] = jnp.maximum(agg + r + br_ref[...], 0.0)


def _tc_finalize(parts, x, W_root, b_root):
    grid = (N_NODES // TN,)
    return pl.pallas_call(
        _fin_body,
        grid=grid,
        in_specs=[
            pl.BlockSpec((TN, DIM_OUT), lambda i: (i, 0)),
            pl.BlockSpec((TN, DIM_OUT), lambda i: (i + N_NODES // TN, 0)),
            pl.BlockSpec((TN, DIM_IN), lambda i: (i, 0)),
            pl.BlockSpec((DIM_IN, DIM_OUT), lambda i: (0, 0)),
            pl.BlockSpec((1, DIM_OUT), lambda i: (0, 0)),
        ],
        out_specs=pl.BlockSpec((TN, DIM_OUT), lambda i: (i, 0)),
        out_shape=jax.ShapeDtypeStruct((N_NODES, DIM_OUT), jnp.float32),
    )(parts, parts, x, W_root, b_root.reshape(1, -1))


def kernel(x, edge_index, edge_attr, W1, b1, W2, b2, W_root, b_root):
    src = edge_index[0].astype(jnp.int32)
    dst = edge_index[1].astype(jnp.int32)
    x_j = _sc_gather(x, src)
    xj4 = jnp.reshape(x_j, (N_ROWS, LANES))
    ea4 = jnp.reshape(edge_attr, (N_ROWS, PACK * BOND_FDIM))
    msg4 = _tc_messages(ea4, xj4, W1, b1, W2, b2)
    msg = jnp.reshape(msg4, (N_EDGES, DIM_OUT))
    parts = _sc_scatter(msg, dst)
    return _tc_finalize(parts, x, W_root, b_root)


# speedup vs baseline: 4.3617x; 1.0025x over previous
import numpy as np
import jax
import jax.numpy as jnp
from jax import lax
from jax.experimental import pallas as pl
from jax.experimental.pallas import tpu as pltpu
from jax.experimental.pallas import tpu_sc as plsc

N_NODES = 10000
N_EDGES = 160000
DIM_IN = 32
DIM_OUT = 32
DIM_HID = 64
BOND_FDIM = 16

NC = 2
NS = 16
NW = NC * NS
CHUNK = 128
PACK = 4
LANES = PACK * DIM_IN
N_CHUNKS = N_EDGES // CHUNK
STEADY = N_CHUNKS // NW
EXTRA_W = N_CHUNKS - STEADY * NW

_SC_MESH = dict(core_axis_name="c", subcore_axis_name="s")
_SC_PARAMS = pltpu.CompilerParams(use_tc_tiling_on_sc=False)


def _load_all_idx(src_hbm, idx_all, wid, sem):
    def fire(k, carry):
        off = (wid + k * NW) * CHUNK
        pltpu.make_async_copy(src_hbm.at[pl.ds(off, CHUNK)],
                              idx_all.at[k], sem).start()
        return carry

    lax.fori_loop(0, STEADY, fire, 0)

    def drain(k, carry):
        pltpu.make_async_copy(src_hbm.at[pl.ds(0, CHUNK)],
                              idx_all.at[k], sem).wait()
        return carry

    lax.fori_loop(0, STEADY, drain, 0)


def _gather_body(x_hbm, src_hbm, out4_hbm, idx_all, rows0, rows1, sem,
                 g0, g1):
    c = lax.axis_index("c")
    s = lax.axis_index("s")
    wid = s * NC + c
    _load_all_idx(src_hbm, idx_all, wid, sem)
    rows = (rows0, rows1)
    gsem = (g0, g1)

    def start(k, b):
        pltpu.make_async_copy(x_hbm.at[idx_all.at[k]], rows[b], gsem[b]).start()

    def finish(k, b):
        pltpu.make_async_copy(x_hbm.at[idx_all.at[k]], rows[b], gsem[b]).wait()
        off = (wid + k * NW) * CHUNK
        pltpu.sync_copy(rows[b], out4_hbm.at[pl.ds(off, CHUNK)])

    start(0, 0)

    def body(kp, carry):
        k = 2 * kp
        start(k + 1, 1)
        finish(k, 0)
        start(k + 2, 0)
        finish(k + 1, 1)
        return carry

    lax.fori_loop(0, (STEADY - 1) // 2, body, 0)
    finish(STEADY - 1, 0)

    @pl.when(wid < EXTRA_W)
    def _():
        pltpu.sync_copy(src_hbm.at[pl.ds((wid + STEADY * NW) * CHUNK, CHUNK)],
                        idx_all.at[0])
        pltpu.async_copy(x_hbm.at[idx_all.at[0]], rows0, g0).wait()
        pltpu.sync_copy(rows0, out4_hbm.at[pl.ds((wid + STEADY * NW) * CHUNK, CHUNK)])


def _sc_gather(x, src):
    kern = pl.kernel(
        _gather_body,
        out_type=jax.ShapeDtypeStruct((N_EDGES, DIM_IN), jnp.float32),
        mesh=plsc.VectorSubcoreMesh(**_SC_MESH),
        scratch_types=[
            pltpu.VMEM((STEADY, CHUNK), jnp.int32),
            pltpu.VMEM((CHUNK, DIM_IN), jnp.float32),
            pltpu.VMEM((CHUNK, DIM_IN), jnp.float32),
            pltpu.SemaphoreType.DMA,
            pltpu.SemaphoreType.DMA,
            pltpu.SemaphoreType.DMA,
        ],
        compiler_params=_SC_PARAMS,
    )
    return kern(x, src)


INIT_ROWS = (N_NODES // NS) // 8 * 8
INIT_REM = N_NODES - INIT_ROWS * NS


def _scatter_body(msg4_hbm, dst_hbm, zeros_hbm, out_hbm, idx_all,
                  rows0, rows1, acc, sem, g0, g1):
    c = lax.axis_index("c")
    s = lax.axis_index("s")
    wid = s * NC + c
    row0 = s * INIT_ROWS

    pltpu.sync_copy(zeros_hbm.at[pl.ds(0, INIT_ROWS)], acc.at[pl.ds(row0, INIT_ROWS)])

    @pl.when(s == 0)
    def _():
        pltpu.sync_copy(zeros_hbm.at[pl.ds(0, INIT_REM)],
                        acc.at[pl.ds(NS * INIT_ROWS, INIT_REM)])

    _load_all_idx(dst_hbm, idx_all, wid, sem)
    plsc.subcore_barrier()
    rows = (rows0, rows1)
    gsem = (g0, g1)

    def start(k, b):
        off = (wid + k * NW) * CHUNK
        pltpu.make_async_copy(msg4_hbm.at[pl.ds(off, CHUNK)],
                              rows[b], gsem[b]).start()

    def finish(k, b):
        off = (wid + k * NW) * CHUNK
        pltpu.make_async_copy(msg4_hbm.at[pl.ds(off, CHUNK)],
                              rows[b], gsem[b]).wait()
        pltpu.sync_copy(rows[b], acc.at[idx_all.at[k]], add=True)

    start(0, 0)

    def body(kp, carry):
        k = 2 * kp
        start(k + 1, 1)
        finish(k, 0)
        start(k + 2, 0)
        finish(k + 1, 1)
        return carry

    lax.fori_loop(0, (STEADY - 1) // 2, body, 0)
    finish(STEADY - 1, 0)

    @pl.when(wid < EXTRA_W)
    def _():
        pltpu.sync_copy(dst_hbm.at[pl.ds((wid + STEADY * NW) * CHUNK, CHUNK)],
                        idx_all.at[0])
        pltpu.sync_copy(msg4_hbm.at[pl.ds((wid + STEADY * NW) * CHUNK, CHUNK)], rows0)
        pltpu.sync_copy(rows0, acc.at[idx_all.at[0]], add=True)

    plsc.subcore_barrier()

    pltpu.sync_copy(acc.at[pl.ds(row0, INIT_ROWS)],
                    out_hbm.at[pl.ds(c * N_NODES + row0, INIT_ROWS)])

    @pl.when(s == 0)
    def _():
        pltpu.sync_copy(acc.at[pl.ds(NS * INIT_ROWS, INIT_REM)],
                        out_hbm.at[pl.ds(c * N_NODES + NS * INIT_ROWS, INIT_REM)])


def _sc_scatter(msg, dst):
    zeros = jnp.zeros((INIT_ROWS, DIM_OUT), jnp.float32)
    kern = pl.kernel(
        _scatter_body,
        out_type=jax.ShapeDtypeStruct((NC * N_NODES, DIM_OUT), jnp.float32),
        mesh=plsc.VectorSubcoreMesh(**_SC_MESH),
        scratch_types=[
            pltpu.VMEM((STEADY, CHUNK), jnp.int32),
            pltpu.VMEM((CHUNK, DIM_OUT), jnp.float32),
            pltpu.VMEM((CHUNK, DIM_OUT), jnp.float32),
            pltpu.VMEM_SHARED((N_NODES, DIM_OUT), jnp.float32),
            pltpu.SemaphoreType.DMA,
            pltpu.SemaphoreType.DMA,
            pltpu.SemaphoreType.DMA,
        ],
        compiler_params=_SC_PARAMS,
    )
    return kern(msg, dst, zeros)


TE = 1600
TR = TE // PACK
N_ROWS = N_EDGES // PACK
HID4 = PACK * DIM_HID
UW = DIM_IN * DIM_OUT
UW4 = PACK * UW


def _msg_body(ea_ref, xj_ref, w1_ref, b1_ref, w2_ref, b2_ref, rep_ref, o_ref):
    h = jnp.dot(ea_ref[...], w1_ref[...], preferred_element_type=jnp.float32)
    h = jnp.maximum(h + b1_ref[...], 0.0)
    u = jnp.dot(h.astype(jnp.bfloat16), w2_ref[...],
                preferred_element_type=jnp.float32)
    u = u + b2_ref[...]
    xb = jnp.dot(xj_ref[...], rep_ref[...], preferred_element_type=jnp.float32)
    p = u * xb
    blocks = []
    for q in range(PACK):
        b = p[:, q * UW:(q + 1) * UW]
        w = UW // 2
        while w >= DIM_OUT:
            b = b[:, :w] + b[:, w:]
            w //= 2
        blocks.append(b)
    o_ref[...] = jnp.concatenate(blocks, axis=1)


def _tc_messages(ea4, xj4, W1, b1, W2, b2):
    eye = jnp.eye(PACK, dtype=jnp.float32)
    w1bd = jnp.kron(eye, W1)
    w2bd = jnp.kron(eye, W2).astype(jnp.bfloat16)
    b1_4 = jnp.tile(b1, PACK).reshape(1, HID4)
    b2_4 = jnp.tile(b2, PACK).reshape(1, UW4)
    rep = np.kron(np.eye(PACK, dtype=np.float32),
                  np.kron(np.eye(DIM_IN, dtype=np.float32),
                          np.ones((1, DIM_OUT), dtype=np.float32)))
    rep = jnp.asarray(rep)
    grid = (N_ROWS // TR,)
    return pl.pallas_call(
        _msg_body,
        grid=grid,
        in_specs=[
            pl.BlockSpec((TR, PACK * BOND_FDIM), lambda i: (i, 0)),
            pl.BlockSpec((TR, LANES), lambda i: (i, 0)),
            pl.BlockSpec((DIM_HID, HID4), lambda i: (0, 0)),
            pl.BlockSpec((1, HID4), lambda i: (0, 0)),
            pl.BlockSpec((HID4, UW4), lambda i: (0, 0)),
            pl.BlockSpec((1, UW4), lambda i: (0, 0)),
            pl.BlockSpec((LANES, UW4), lambda i: (0, 0)),
        ],
        out_specs=pl.BlockSpec((TR, LANES), lambda i: (i, 0)),
        out_shape=jax.ShapeDtypeStruct((N_ROWS, LANES), jnp.float32),
    )(ea4, xj4, w1bd, b1_4, w2bd, b2_4, rep)


TN = 1000


def _fin_body(a0_ref, a1_ref, x_ref, wr_ref, br_ref, o_ref):
    r = jnp.dot(x_ref[...], wr_ref[...], preferred_element_type=jnp.float32)
    agg = a0_ref[...] + a1_ref[...]
    o_ref[...] = jnp.maximum(agg + r + br_ref[...], 0.0)


def _tc_finalize(parts, x, W_root, b_root):
    grid = (N_NODES // TN,)
    return pl.pallas_call(
        _fin_body,
        grid=grid,
        in_specs=[
            pl.BlockSpec((TN, DIM_OUT), lambda i: (i, 0)),
            pl.BlockSpec((TN, DIM_OUT), lambda i: (i + N_NODES // TN, 0)),
            pl.BlockSpec((TN, DIM_IN), lambda i: (i, 0)),
            pl.BlockSpec((DIM_IN, DIM_OUT), lambda i: (0, 0)),
            pl.BlockSpec((1, DIM_OUT), lambda i: (0, 0)),
        ],
        out_specs=pl.BlockSpec((TN, DIM_OUT), lambda i: (i, 0)),
        out_shape=jax.ShapeDtypeStruct((N_NODES, DIM_OUT), jnp.float32),
    )(parts, parts, x, W_root, b_root.reshape(1, -1))


def kernel(x, edge_index, edge_attr, W1, b1, W2, b2, W_root, b_root):
    src = edge_index[0].astype(jnp.int32)
    dst = edge_index[1].astype(jnp.int32)
    x_j = _sc_gather(x, src)
    xj4 = jnp.reshape(x_j, (N_ROWS, LANES))
    ea4 = jnp.reshape(edge_attr, (N_ROWS, PACK * BOND_FDIM))
    msg4 = _tc_messages(ea4, xj4, W1, b1, W2, b2)
    msg = jnp.reshape(msg4, (N_EDGES, DIM_OUT))
    parts = _sc_scatter(msg, dst)
    return _tc_finalize(parts, x, W_root, b_root)


# speedup vs baseline: 4.6479x; 1.0656x over previous
import numpy as np
import jax
import jax.numpy as jnp
from jax import lax
from jax.experimental import pallas as pl
from jax.experimental.pallas import tpu as pltpu
from jax.experimental.pallas import tpu_sc as plsc

N_NODES = 10000
N_EDGES = 160000
DIM_IN = 32
DIM_OUT = 32
DIM_HID = 64
BOND_FDIM = 16

NC = 2
NS = 16
NW = NC * NS
CHUNK = 128
PACK = 4
LANES = PACK * DIM_IN
N_CHUNKS = N_EDGES // CHUNK
STEADY = N_CHUNKS // NW
EXTRA_W = N_CHUNKS - STEADY * NW

_SC_MESH = dict(core_axis_name="c", subcore_axis_name="s")
_SC_PARAMS = pltpu.CompilerParams(use_tc_tiling_on_sc=False)


def _load_all_idx(src_hbm, idx_all, wid, sem):
    def fire(k, carry):
        off = (wid + k * NW) * CHUNK
        pltpu.make_async_copy(src_hbm.at[pl.ds(off, CHUNK)],
                              idx_all.at[k], sem).start()
        return carry

    lax.fori_loop(0, STEADY, fire, 0)

    def drain(k, carry):
        pltpu.make_async_copy(src_hbm.at[pl.ds(0, CHUNK)],
                              idx_all.at[k], sem).wait()
        return carry

    lax.fori_loop(0, STEADY, drain, 0)


def _gather_body(x_hbm, src_hbm, out4_hbm, idx_all, rows0, rows1, sem,
                 g0, g1):
    c = lax.axis_index("c")
    s = lax.axis_index("s")
    wid = s * NC + c
    _load_all_idx(src_hbm, idx_all, wid, sem)
    rows = (rows0, rows1)
    gsem = (g0, g1)

    def start(k, b):
        pltpu.make_async_copy(x_hbm.at[idx_all.at[k]], rows[b], gsem[b]).start()

    def finish(k, b):
        pltpu.make_async_copy(x_hbm.at[idx_all.at[k]], rows[b], gsem[b]).wait()
        off = (wid + k * NW) * CHUNK
        pltpu.sync_copy(rows[b], out4_hbm.at[pl.ds(off, CHUNK)])

    start(0, 0)

    def body(kp, carry):
        k = 2 * kp
        start(k + 1, 1)
        finish(k, 0)
        start(k + 2, 0)
        finish(k + 1, 1)
        return carry

    lax.fori_loop(0, (STEADY - 1) // 2, body, 0)
    finish(STEADY - 1, 0)

    @pl.when(wid < EXTRA_W)
    def _():
        pltpu.sync_copy(src_hbm.at[pl.ds((wid + STEADY * NW) * CHUNK, CHUNK)],
                        idx_all.at[0])
        pltpu.async_copy(x_hbm.at[idx_all.at[0]], rows0, g0).wait()
        pltpu.sync_copy(rows0, out4_hbm.at[pl.ds((wid + STEADY * NW) * CHUNK, CHUNK)])


def _sc_gather(x, src):
    kern = pl.kernel(
        _gather_body,
        out_type=jax.ShapeDtypeStruct((N_EDGES, DIM_IN), jnp.float32),
        mesh=plsc.VectorSubcoreMesh(**_SC_MESH),
        scratch_types=[
            pltpu.VMEM((STEADY, CHUNK), jnp.int32),
            pltpu.VMEM((CHUNK, DIM_IN), jnp.float32),
            pltpu.VMEM((CHUNK, DIM_IN), jnp.float32),
            pltpu.SemaphoreType.DMA,
            pltpu.SemaphoreType.DMA,
            pltpu.SemaphoreType.DMA,
        ],
        compiler_params=_SC_PARAMS,
    )
    return kern(x, src)


INIT_ROWS = (N_NODES // NS) // 8 * 8
INIT_REM = N_NODES - INIT_ROWS * NS


def _scatter_body(msg4_hbm, dst_hbm, zeros_hbm, out_hbm, idx_all,
                  rows0, rows1, acc, sem, g0, g1):
    c = lax.axis_index("c")
    s = lax.axis_index("s")
    wid = s * NC + c
    row0 = s * INIT_ROWS

    pltpu.sync_copy(zeros_hbm.at[pl.ds(0, INIT_ROWS)], acc.at[pl.ds(row0, INIT_ROWS)])

    @pl.when(s == 0)
    def _():
        pltpu.sync_copy(zeros_hbm.at[pl.ds(0, INIT_REM)],
                        acc.at[pl.ds(NS * INIT_ROWS, INIT_REM)])

    _load_all_idx(dst_hbm, idx_all, wid, sem)
    plsc.subcore_barrier()
    rows = (rows0, rows1)
    gsem = (g0, g1)

    def start(k, b):
        off = (wid + k * NW) * CHUNK
        pltpu.make_async_copy(msg4_hbm.at[pl.ds(off, CHUNK)],
                              rows[b], gsem[b]).start()

    def finish(k, b):
        off = (wid + k * NW) * CHUNK
        pltpu.make_async_copy(msg4_hbm.at[pl.ds(off, CHUNK)],
                              rows[b], gsem[b]).wait()
        pltpu.sync_copy(rows[b], acc.at[idx_all.at[k]], add=True)

    start(0, 0)

    def body(kp, carry):
        k = 2 * kp
        start(k + 1, 1)
        finish(k, 0)
        start(k + 2, 0)
        finish(k + 1, 1)
        return carry

    lax.fori_loop(0, (STEADY - 1) // 2, body, 0)
    finish(STEADY - 1, 0)

    @pl.when(wid < EXTRA_W)
    def _():
        pltpu.sync_copy(dst_hbm.at[pl.ds((wid + STEADY * NW) * CHUNK, CHUNK)],
                        idx_all.at[0])
        pltpu.sync_copy(msg4_hbm.at[pl.ds((wid + STEADY * NW) * CHUNK, CHUNK)], rows0)
        pltpu.sync_copy(rows0, acc.at[idx_all.at[0]], add=True)

    plsc.subcore_barrier()

    pltpu.sync_copy(acc.at[pl.ds(row0, INIT_ROWS)],
                    out_hbm.at[pl.ds(c * N_NODES + row0, INIT_ROWS)])

    @pl.when(s == 0)
    def _():
        pltpu.sync_copy(acc.at[pl.ds(NS * INIT_ROWS, INIT_REM)],
                        out_hbm.at[pl.ds(c * N_NODES + NS * INIT_ROWS, INIT_REM)])


def _sc_scatter(msg, dst):
    zeros = jnp.zeros((INIT_ROWS, DIM_OUT), jnp.float32)
    kern = pl.kernel(
        _scatter_body,
        out_type=jax.ShapeDtypeStruct((NC * N_NODES, DIM_OUT), jnp.float32),
        mesh=plsc.VectorSubcoreMesh(**_SC_MESH),
        scratch_types=[
            pltpu.VMEM((STEADY, CHUNK), jnp.int32),
            pltpu.VMEM((CHUNK, DIM_OUT), jnp.float32),
            pltpu.VMEM((CHUNK, DIM_OUT), jnp.float32),
            pltpu.VMEM_SHARED((N_NODES, DIM_OUT), jnp.float32),
            pltpu.SemaphoreType.DMA,
            pltpu.SemaphoreType.DMA,
            pltpu.SemaphoreType.DMA,
        ],
        compiler_params=_SC_PARAMS,
    )
    return kern(msg, dst, zeros)


TE = 3200
TR = TE // PACK
N_ROWS = N_EDGES // PACK
HID4 = PACK * DIM_HID
UW = DIM_IN * DIM_OUT
UW4 = PACK * UW


def _msg_body(ea_ref, xj_ref, w1_ref, b1_ref, w2_ref, b2_ref, rep_ref, o_ref):
    h = jnp.dot(ea_ref[...], w1_ref[...], preferred_element_type=jnp.float32)
    h = jnp.maximum(h + b1_ref[...], 0.0)
    u = jnp.dot(h.astype(jnp.bfloat16), w2_ref[...],
                preferred_element_type=jnp.float32)
    u = u + b2_ref[...]
    xb = jnp.dot(xj_ref[...].astype(jnp.bfloat16), rep_ref[...],
                 preferred_element_type=jnp.float32)
    p = u * xb
    blocks = []
    for q in range(PACK):
        b = p[:, q * UW:(q + 1) * UW]
        w = UW // 2
        while w >= DIM_OUT:
            b = b[:, :w] + b[:, w:]
            w //= 2
        blocks.append(b)
    o_ref[...] = jnp.concatenate(blocks, axis=1)


def _tc_messages(ea4, xj4, W1, b1, W2, b2):
    eye = jnp.eye(PACK, dtype=jnp.float32)
    w1bd = jnp.kron(eye, W1)
    w2bd = jnp.kron(eye, W2).astype(jnp.bfloat16)
    b1_4 = jnp.tile(b1, PACK).reshape(1, HID4)
    b2_4 = jnp.tile(b2, PACK).reshape(1, UW4)
    rep = np.kron(np.eye(PACK, dtype=np.float32),
                  np.kron(np.eye(DIM_IN, dtype=np.float32),
                          np.ones((1, DIM_OUT), dtype=np.float32)))
    rep = jnp.asarray(rep, dtype=jnp.bfloat16)
    grid = (N_ROWS // TR,)
    return pl.pallas_call(
        _msg_body,
        grid=grid,
        in_specs=[
            pl.BlockSpec((TR, PACK * BOND_FDIM), lambda i: (i, 0)),
            pl.BlockSpec((TR, LANES), lambda i: (i, 0)),
            pl.BlockSpec((DIM_HID, HID4), lambda i: (0, 0)),
            pl.BlockSpec((1, HID4), lambda i: (0, 0)),
            pl.BlockSpec((HID4, UW4), lambda i: (0, 0)),
            pl.BlockSpec((1, UW4), lambda i: (0, 0)),
            pl.BlockSpec((LANES, UW4), lambda i: (0, 0)),
        ],
        out_specs=pl.BlockSpec((TR, LANES), lambda i: (i, 0)),
        out_shape=jax.ShapeDtypeStruct((N_ROWS, LANES), jnp.float32),
    )(ea4, xj4, w1bd, b1_4, w2bd, b2_4, rep)


TN = 1000


def _fin_body(a0_ref, a1_ref, x_ref, wr_ref, br_ref, o_ref):
    r = jnp.dot(x_ref[...], wr_ref[...], preferred_element_type=jnp.float32)
    agg = a0_ref[...] + a1_ref[...]
    o_ref[...] = jnp.maximum(agg + r + br_ref[...], 0.0)


def _tc_finalize(parts, x, W_root, b_root):
    grid = (N_NODES // TN,)
    return pl.pallas_call(
        _fin_body,
        grid=grid,
        in_specs=[
            pl.BlockSpec((TN, DIM_OUT), lambda i: (i, 0)),
            pl.BlockSpec((TN, DIM_OUT), lambda i: (i + N_NODES // TN, 0)),
            pl.BlockSpec((TN, DIM_IN), lambda i: (i, 0)),
            pl.BlockSpec((DIM_IN, DIM_OUT), lambda i: (0, 0)),
            pl.BlockSpec((1, DIM_OUT), lambda i: (0, 0)),
        ],
        out_specs=pl.BlockSpec((TN, DIM_OUT), lambda i: (i, 0)),
        out_shape=jax.ShapeDtypeStruct((N_NODES, DIM_OUT), jnp.float32),
    )(parts, parts, x, W_root, b_root.reshape(1, -1))


def kernel(x, edge_index, edge_attr, W1, b1, W2, b2, W_root, b_root):
    src = edge_index[0].astype(jnp.int32)
    dst = edge_index[1].astype(jnp.int32)
    x_j = _sc_gather(x, src)
    xj4 = jnp.reshape(x_j, (N_ROWS, LANES))
    ea4 = jnp.reshape(edge_attr, (N_ROWS, PACK * BOND_FDIM))
    msg4 = _tc_messages(ea4, xj4, W1, b1, W2, b2)
    msg = jnp.reshape(msg4, (N_EDGES, DIM_OUT))
    parts = _sc_scatter(msg, dst)
    return _tc_finalize(parts, x, W_root, b_root)


# speedup vs baseline: 4.6862x; 1.0082x over previous
import numpy as np
import jax
import jax.numpy as jnp
from jax import lax
from jax.experimental import pallas as pl
from jax.experimental.pallas import tpu as pltpu
from jax.experimental.pallas import tpu_sc as plsc

N_NODES = 10000
N_EDGES = 160000
DIM_IN = 32
DIM_OUT = 32
DIM_HID = 64
BOND_FDIM = 16

NC = 2
NS = 16
NW = NC * NS
CHUNK = 128
PACK = 4
LANES = PACK * DIM_IN
N_CHUNKS = N_EDGES // CHUNK
STEADY = N_CHUNKS // NW
EXTRA_W = N_CHUNKS - STEADY * NW

_SC_MESH = dict(core_axis_name="c", subcore_axis_name="s")
_SC_PARAMS = pltpu.CompilerParams(use_tc_tiling_on_sc=False)


def _load_all_idx(src_hbm, idx_all, wid, sem):
    def fire(k, carry):
        off = (wid + k * NW) * CHUNK
        pltpu.make_async_copy(src_hbm.at[pl.ds(off, CHUNK)],
                              idx_all.at[k], sem).start()
        return carry

    lax.fori_loop(0, STEADY, fire, 0)

    def drain(k, carry):
        pltpu.make_async_copy(src_hbm.at[pl.ds(0, CHUNK)],
                              idx_all.at[k], sem).wait()
        return carry

    lax.fori_loop(0, STEADY, drain, 0)


def _gather_body(x_hbm, src_hbm, out4_hbm, idx_all, rows0, rows1, sem,
                 g0, g1):
    c = lax.axis_index("c")
    s = lax.axis_index("s")
    wid = s * NC + c
    _load_all_idx(src_hbm, idx_all, wid, sem)
    rows = (rows0, rows1)
    gsem = (g0, g1)

    def start(k, b):
        pltpu.make_async_copy(x_hbm.at[idx_all.at[k]], rows[b], gsem[b]).start()

    def finish(k, b):
        pltpu.make_async_copy(x_hbm.at[idx_all.at[k]], rows[b], gsem[b]).wait()
        off = (wid + k * NW) * CHUNK
        pltpu.sync_copy(rows[b], out4_hbm.at[pl.ds(off, CHUNK)])

    start(0, 0)

    def body(kp, carry):
        k = 2 * kp
        start(k + 1, 1)
        finish(k, 0)
        start(k + 2, 0)
        finish(k + 1, 1)
        return carry

    lax.fori_loop(0, (STEADY - 1) // 2, body, 0)
    finish(STEADY - 1, 0)

    @pl.when(wid < EXTRA_W)
    def _():
        pltpu.sync_copy(src_hbm.at[pl.ds((wid + STEADY * NW) * CHUNK, CHUNK)],
                        idx_all.at[0])
        pltpu.async_copy(x_hbm.at[idx_all.at[0]], rows0, g0).wait()
        pltpu.sync_copy(rows0, out4_hbm.at[pl.ds((wid + STEADY * NW) * CHUNK, CHUNK)])


def _sc_gather(x, src):
    kern = pl.kernel(
        _gather_body,
        out_type=jax.ShapeDtypeStruct((N_EDGES, DIM_IN), jnp.float32),
        mesh=plsc.VectorSubcoreMesh(**_SC_MESH),
        scratch_types=[
            pltpu.VMEM((STEADY, CHUNK), jnp.int32),
            pltpu.VMEM((CHUNK, DIM_IN), jnp.float32),
            pltpu.VMEM((CHUNK, DIM_IN), jnp.float32),
            pltpu.SemaphoreType.DMA,
            pltpu.SemaphoreType.DMA,
            pltpu.SemaphoreType.DMA,
        ],
        compiler_params=_SC_PARAMS,
    )
    return kern(x, src)


INIT_ROWS = (N_NODES // NS) // 8 * 8
INIT_REM = N_NODES - INIT_ROWS * NS


def _scatter_body(msg4_hbm, dst_hbm, zeros_hbm, out_hbm, idx_all,
                  rows0, rows1, acc, sem, g0, g1):
    c = lax.axis_index("c")
    s = lax.axis_index("s")
    wid = s * NC + c
    row0 = s * INIT_ROWS

    pltpu.sync_copy(zeros_hbm.at[pl.ds(0, INIT_ROWS)], acc.at[pl.ds(row0, INIT_ROWS)])

    @pl.when(s == 0)
    def _():
        pltpu.sync_copy(zeros_hbm.at[pl.ds(0, INIT_REM)],
                        acc.at[pl.ds(NS * INIT_ROWS, INIT_REM)])

    _load_all_idx(dst_hbm, idx_all, wid, sem)
    plsc.subcore_barrier()
    rows = (rows0, rows1)
    gsem = (g0, g1)

    def start(k, b):
        off = (wid + k * NW) * CHUNK
        pltpu.make_async_copy(msg4_hbm.at[pl.ds(off, CHUNK)],
                              rows[b], gsem[b]).start()

    def finish(k, b):
        off = (wid + k * NW) * CHUNK
        pltpu.make_async_copy(msg4_hbm.at[pl.ds(off, CHUNK)],
                              rows[b], gsem[b]).wait()
        pltpu.sync_copy(rows[b], acc.at[idx_all.at[k]], add=True)

    start(0, 0)

    def body(kp, carry):
        k = 2 * kp
        start(k + 1, 1)
        finish(k, 0)
        start(k + 2, 0)
        finish(k + 1, 1)
        return carry

    lax.fori_loop(0, (STEADY - 1) // 2, body, 0)
    finish(STEADY - 1, 0)

    @pl.when(wid < EXTRA_W)
    def _():
        pltpu.sync_copy(dst_hbm.at[pl.ds((wid + STEADY * NW) * CHUNK, CHUNK)],
                        idx_all.at[0])
        pltpu.sync_copy(msg4_hbm.at[pl.ds((wid + STEADY * NW) * CHUNK, CHUNK)], rows0)
        pltpu.sync_copy(rows0, acc.at[idx_all.at[0]], add=True)

    plsc.subcore_barrier()

    pltpu.sync_copy(acc.at[pl.ds(row0, INIT_ROWS)],
                    out_hbm.at[pl.ds(c * N_NODES + row0, INIT_ROWS)])

    @pl.when(s == 0)
    def _():
        pltpu.sync_copy(acc.at[pl.ds(NS * INIT_ROWS, INIT_REM)],
                        out_hbm.at[pl.ds(c * N_NODES + NS * INIT_ROWS, INIT_REM)])


def _sc_scatter(msg, dst):
    zeros = jnp.zeros((INIT_ROWS, DIM_OUT), jnp.float32)
    kern = pl.kernel(
        _scatter_body,
        out_type=jax.ShapeDtypeStruct((NC * N_NODES, DIM_OUT), jnp.float32),
        mesh=plsc.VectorSubcoreMesh(**_SC_MESH),
        scratch_types=[
            pltpu.VMEM((STEADY, CHUNK), jnp.int32),
            pltpu.VMEM((CHUNK, DIM_OUT), jnp.float32),
            pltpu.VMEM((CHUNK, DIM_OUT), jnp.float32),
            pltpu.VMEM_SHARED((N_NODES, DIM_OUT), jnp.float32),
            pltpu.SemaphoreType.DMA,
            pltpu.SemaphoreType.DMA,
            pltpu.SemaphoreType.DMA,
        ],
        compiler_params=_SC_PARAMS,
    )
    return kern(msg, dst, zeros)


TE = 4000
TR = TE // PACK
N_ROWS = N_EDGES // PACK
HID4 = PACK * DIM_HID
UW = DIM_IN * DIM_OUT
UW4 = PACK * UW


def _msg_body(ea_ref, xj_ref, w1_ref, b1_ref, w2_ref, b2_ref, rep_ref, o_ref):
    h = jnp.dot(ea_ref[...], w1_ref[...], preferred_element_type=jnp.float32)
    h = jnp.maximum(h + b1_ref[...], 0.0)
    u = jnp.dot(h.astype(jnp.bfloat16), w2_ref[...],
                preferred_element_type=jnp.float32)
    u = u + b2_ref[...]
    xb = jnp.dot(xj_ref[...].astype(jnp.bfloat16), rep_ref[...],
                 preferred_element_type=jnp.float32)
    p = u * xb
    blocks = []
    for q in range(PACK):
        b = p[:, q * UW:(q + 1) * UW]
        w = UW // 2
        while w >= DIM_OUT:
            b = b[:, :w] + b[:, w:]
            w //= 2
        blocks.append(b)
    o_ref[...] = jnp.concatenate(blocks, axis=1)


def _tc_messages(ea4, xj4, W1, b1, W2, b2):
    eye = jnp.eye(PACK, dtype=jnp.float32)
    w1bd = jnp.kron(eye, W1)
    w2bd = jnp.kron(eye, W2).astype(jnp.bfloat16)
    b1_4 = jnp.tile(b1, PACK).reshape(1, HID4)
    b2_4 = jnp.tile(b2, PACK).reshape(1, UW4)
    rep = np.kron(np.eye(PACK, dtype=np.float32),
                  np.kron(np.eye(DIM_IN, dtype=np.float32),
                          np.ones((1, DIM_OUT), dtype=np.float32)))
    rep = jnp.asarray(rep, dtype=jnp.bfloat16)
    grid = (N_ROWS // TR,)
    return pl.pallas_call(
        _msg_body,
        grid=grid,
        in_specs=[
            pl.BlockSpec((TR, PACK * BOND_FDIM), lambda i: (i, 0)),
            pl.BlockSpec((TR, LANES), lambda i: (i, 0)),
            pl.BlockSpec((DIM_HID, HID4), lambda i: (0, 0)),
            pl.BlockSpec((1, HID4), lambda i: (0, 0)),
            pl.BlockSpec((HID4, UW4), lambda i: (0, 0)),
            pl.BlockSpec((1, UW4), lambda i: (0, 0)),
            pl.BlockSpec((LANES, UW4), lambda i: (0, 0)),
        ],
        out_specs=pl.BlockSpec((TR, LANES), lambda i: (i, 0)),
        out_shape=jax.ShapeDtypeStruct((N_ROWS, LANES), jnp.float32),
    )(ea4, xj4, w1bd, b1_4, w2bd, b2_4, rep)


TN = 1000


def _fin_body(a0_ref, a1_ref, x_ref, wr_ref, br_ref, o_ref):
    r = jnp.dot(x_ref[...], wr_ref[...], preferred_element_type=jnp.float32)
    agg = a0_ref[...] + a1_ref[...]
    o_ref[...] = jnp.maximum(agg + r + br_ref[...], 0.0)


def _tc_finalize(parts, x, W_root, b_root):
    grid = (N_NODES // TN,)
    return pl.pallas_call(
        _fin_body,
        grid=grid,
        in_specs=[
            pl.BlockSpec((TN, DIM_OUT), lambda i: (i, 0)),
            pl.BlockSpec((TN, DIM_OUT), lambda i: (i + N_NODES // TN, 0)),
            pl.BlockSpec((TN, DIM_IN), lambda i: (i, 0)),
            pl.BlockSpec((DIM_IN, DIM_OUT), lambda i: (0, 0)),
            pl.BlockSpec((1, DIM_OUT), lambda i: (0, 0)),
        ],
        out_specs=pl.BlockSpec((TN, DIM_OUT), lambda i: (i, 0)),
        out_shape=jax.ShapeDtypeStruct((N_NODES, DIM_OUT), jnp.float32),
    )(parts, parts, x, W_root, b_root.reshape(1, -1))


def kernel(x, edge_index, edge_attr, W1, b1, W2, b2, W_root, b_root):
    src = edge_index[0].astype(jnp.int32)
    dst = edge_index[1].astype(jnp.int32)
    x_j = _sc_gather(x, src)
    xj4 = jnp.reshape(x_j, (N_ROWS, LANES))
    ea4 = jnp.reshape(edge_attr, (N_ROWS, PACK * BOND_FDIM))
    msg4 = _tc_messages(ea4, xj4, W1, b1, W2, b2)
    msg = jnp.reshape(msg4, (N_EDGES, DIM_OUT))
    parts = _sc_scatter(msg, dst)
    return _tc_finalize(parts, x, W_root, b_root)


# speedup vs baseline: 4.6870x; 1.0002x over previous
import numpy as np
import jax
import jax.numpy as jnp
from jax import lax
from jax.experimental import pallas as pl
from jax.experimental.pallas import tpu as pltpu
from jax.experimental.pallas import tpu_sc as plsc

N_NODES = 10000
N_EDGES = 160000
DIM_IN = 32
DIM_OUT = 32
DIM_HID = 64
BOND_FDIM = 16

NC = 2
NS = 16
NW = NC * NS
CHUNK = 128
PACK = 4
LANES = PACK * DIM_IN
N_CHUNKS = N_EDGES // CHUNK
STEADY = N_CHUNKS // NW
EXTRA_W = N_CHUNKS - STEADY * NW

_SC_MESH = dict(core_axis_name="c", subcore_axis_name="s")
_SC_PARAMS = pltpu.CompilerParams(use_tc_tiling_on_sc=False)


def _load_all_idx(src_hbm, idx_all, wid, sem):
    def fire(k, carry):
        off = (wid + k * NW) * CHUNK
        pltpu.make_async_copy(src_hbm.at[pl.ds(off, CHUNK)],
                              idx_all.at[k], sem).start()
        return carry

    lax.fori_loop(0, STEADY, fire, 0)

    def drain(k, carry):
        pltpu.make_async_copy(src_hbm.at[pl.ds(0, CHUNK)],
                              idx_all.at[k], sem).wait()
        return carry

    lax.fori_loop(0, STEADY, drain, 0)


def _gather_body(x_hbm, src_hbm, out_hbm, idx_all, rows0, rows1, sem,
                 g0, g1):
    c = lax.axis_index("c")
    s = lax.axis_index("s")
    wid = s * NC + c
    _load_all_idx(src_hbm, idx_all, wid, sem)
    rows = (rows0, rows1)
    gsem = (g0, g1)

    def start(k, b):
        pltpu.make_async_copy(x_hbm.at[idx_all.at[k]], rows[b], gsem[b]).start()

    def finish(k, b):
        pltpu.make_async_copy(x_hbm.at[idx_all.at[k]], rows[b], gsem[b]).wait()
        off = (wid + k * NW) * CHUNK
        pltpu.sync_copy(rows[b], out_hbm.at[pl.ds(off, CHUNK)])

    start(0, 0)

    def body(kp, carry):
        k = 2 * kp
        start(k + 1, 1)
        finish(k, 0)
        start(k + 2, 0)
        finish(k + 1, 1)
        return carry

    lax.fori_loop(0, (STEADY - 1) // 2, body, 0)
    finish(STEADY - 1, 0)

    @pl.when(wid < EXTRA_W)
    def _():
        pltpu.sync_copy(src_hbm.at[pl.ds((wid + STEADY * NW) * CHUNK, CHUNK)],
                        idx_all.at[0])
        pltpu.async_copy(x_hbm.at[idx_all.at[0]], rows0, g0).wait()
        pltpu.sync_copy(rows0, out_hbm.at[pl.ds((wid + STEADY * NW) * CHUNK, CHUNK)])


def _sc_gather(x, src):
    kern = pl.kernel(
        _gather_body,
        out_type=jax.ShapeDtypeStruct((N_EDGES, DIM_IN), jnp.float32),
        mesh=plsc.VectorSubcoreMesh(**_SC_MESH),
        scratch_types=[
            pltpu.VMEM((STEADY, CHUNK), jnp.int32),
            pltpu.VMEM((CHUNK, DIM_IN), jnp.float32),
            pltpu.VMEM((CHUNK, DIM_IN), jnp.float32),
            pltpu.SemaphoreType.DMA,
            pltpu.SemaphoreType.DMA,
            pltpu.SemaphoreType.DMA,
        ],
        compiler_params=_SC_PARAMS,
    )
    return kern(x, src)


INIT_ROWS = (N_NODES // NS) // 8 * 8
INIT_REM = N_NODES - INIT_ROWS * NS


def _scatter_body(msg_hbm, dst_hbm, zeros_hbm, out_hbm, idx_all,
                  rows0, rows1, acc, sem, g0, g1):
    c = lax.axis_index("c")
    s = lax.axis_index("s")
    wid = s * NC + c
    row0 = s * INIT_ROWS

    pltpu.sync_copy(zeros_hbm.at[pl.ds(0, INIT_ROWS)], acc.at[pl.ds(row0, INIT_ROWS)])

    @pl.when(s == 0)
    def _():
        pltpu.sync_copy(zeros_hbm.at[pl.ds(0, INIT_REM)],
                        acc.at[pl.ds(NS * INIT_ROWS, INIT_REM)])

    _load_all_idx(dst_hbm, idx_all, wid, sem)
    plsc.subcore_barrier()
    rows = (rows0, rows1)
    gsem = (g0, g1)

    def start(k, b):
        off = (wid + k * NW) * CHUNK
        pltpu.make_async_copy(msg_hbm.at[pl.ds(off, CHUNK)],
                              rows[b], gsem[b]).start()

    def finish(k, b):
        off = (wid + k * NW) * CHUNK
        pltpu.make_async_copy(msg_hbm.at[pl.ds(off, CHUNK)],
                              rows[b], gsem[b]).wait()
        pltpu.sync_copy(rows[b], acc.at[idx_all.at[k]], add=True)

    start(0, 0)

    def body(kp, carry):
        k = 2 * kp
        start(k + 1, 1)
        finish(k, 0)
        start(k + 2, 0)
        finish(k + 1, 1)
        return carry

    lax.fori_loop(0, (STEADY - 1) // 2, body, 0)
    finish(STEADY - 1, 0)

    @pl.when(wid < EXTRA_W)
    def _():
        pltpu.sync_copy(dst_hbm.at[pl.ds((wid + STEADY * NW) * CHUNK, CHUNK)],
                        idx_all.at[0])
        pltpu.sync_copy(msg_hbm.at[pl.ds((wid + STEADY * NW) * CHUNK, CHUNK)], rows0)
        pltpu.sync_copy(rows0, acc.at[idx_all.at[0]], add=True)

    plsc.subcore_barrier()

    pltpu.sync_copy(acc.at[pl.ds(row0, INIT_ROWS)],
                    out_hbm.at[pl.ds(c * N_NODES + row0, INIT_ROWS)])

    @pl.when(s == 0)
    def _():
        pltpu.sync_copy(acc.at[pl.ds(NS * INIT_ROWS, INIT_REM)],
                        out_hbm.at[pl.ds(c * N_NODES + NS * INIT_ROWS, INIT_REM)])


def _sc_scatter(msg, dst):
    zeros = jnp.zeros((INIT_ROWS, DIM_OUT), jnp.float32)
    kern = pl.kernel(
        _scatter_body,
        out_type=jax.ShapeDtypeStruct((NC * N_NODES, DIM_OUT), jnp.float32),
        mesh=plsc.VectorSubcoreMesh(**_SC_MESH),
        scratch_types=[
            pltpu.VMEM((STEADY, CHUNK), jnp.int32),
            pltpu.VMEM((CHUNK, DIM_OUT), jnp.float32),
            pltpu.VMEM((CHUNK, DIM_OUT), jnp.float32),
            pltpu.VMEM_SHARED((N_NODES, DIM_OUT), jnp.float32),
            pltpu.SemaphoreType.DMA,
            pltpu.SemaphoreType.DMA,
            pltpu.SemaphoreType.DMA,
        ],
        compiler_params=_SC_PARAMS,
    )
    return kern(msg, dst, zeros)


TE = 4000
TR = TE // PACK
N_ROWS = N_EDGES // PACK
HID4 = PACK * DIM_HID
UW = DIM_IN * DIM_OUT
UW4 = PACK * UW


def _msg_body(ea_ref, xj_ref, w1_ref, b1_ref, w2_ref, b2_ref, rep_ref, o_ref):
    h = jnp.dot(ea_ref[...], w1_ref[...], preferred_element_type=jnp.float32)
    h = jnp.maximum(h + b1_ref[...], 0.0)
    u = jnp.dot(h.astype(jnp.bfloat16), w2_ref[...],
                preferred_element_type=jnp.float32)
    u = u + b2_ref[...]
    xb = jnp.dot(xj_ref[...].astype(jnp.bfloat16), rep_ref[...],
                 preferred_element_type=jnp.float32)
    p = u * xb
    blocks = []
    for q in range(PACK):
        b = p[:, q * UW:(q + 1) * UW]
        w = UW // 2
        while w >= DIM_OUT:
            b = b[:, :w] + b[:, w:]
            w //= 2
        blocks.append(b)
    o_ref[...] = jnp.concatenate(blocks, axis=1)


def _tc_messages(ea4, xj4, W1, b1, W2, b2):
    eye = jnp.eye(PACK, dtype=jnp.float32)
    w1bd = jnp.kron(eye, W1)
    w2bd = jnp.kron(eye, W2).astype(jnp.bfloat16)
    b1_4 = jnp.tile(b1, PACK).reshape(1, HID4)
    b2_4 = jnp.tile(b2, PACK).reshape(1, UW4)
    rep = np.kron(np.eye(PACK, dtype=np.float32),
                  np.kron(np.eye(DIM_IN, dtype=np.float32),
                          np.ones((1, DIM_OUT), dtype=np.float32)))
    rep = jnp.asarray(rep, dtype=jnp.bfloat16)
    grid = (N_ROWS // TR,)
    return pl.pallas_call(
        _msg_body,
        grid=grid,
        in_specs=[
            pl.BlockSpec((TR, PACK * BOND_FDIM), lambda i: (i, 0)),
            pl.BlockSpec((TR, LANES), lambda i: (i, 0)),
            pl.BlockSpec((DIM_HID, HID4), lambda i: (0, 0)),
            pl.BlockSpec((1, HID4), lambda i: (0, 0)),
            pl.BlockSpec((HID4, UW4), lambda i: (0, 0)),
            pl.BlockSpec((1, UW4), lambda i: (0, 0)),
            pl.BlockSpec((LANES, UW4), lambda i: (0, 0)),
        ],
        out_specs=pl.BlockSpec((TR, LANES), lambda i: (i, 0)),
        out_shape=jax.ShapeDtypeStruct((N_ROWS, LANES), jnp.float32),
    )(ea4, xj4, w1bd, b1_4, w2bd, b2_4, rep)


TN = 1000


def _fin_body(a0_ref, a1_ref, x_ref, wr_ref, br_ref, o_ref):
    r = jnp.dot(x_ref[...], wr_ref[...], preferred_element_type=jnp.float32)
    agg = a0_ref[...] + a1_ref[...]
    o_ref[...] = jnp.maximum(agg + r + br_ref[...], 0.0)


def _tc_finalize(parts, x, W_root, b_root):
    grid = (N_NODES // TN,)
    return pl.pallas_call(
        _fin_body,
        grid=grid,
        in_specs=[
            pl.BlockSpec((TN, DIM_OUT), lambda i: (i, 0)),
            pl.BlockSpec((TN, DIM_OUT), lambda i: (i + N_NODES // TN, 0)),
            pl.BlockSpec((TN, DIM_IN), lambda i: (i, 0)),
            pl.BlockSpec((DIM_IN, DIM_OUT), lambda i: (0, 0)),
            pl.BlockSpec((1, DIM_OUT), lambda i: (0, 0)),
        ],
        out_specs=pl.BlockSpec((TN, DIM_OUT), lambda i: (i, 0)),
        out_shape=jax.ShapeDtypeStruct((N_NODES, DIM_OUT), jnp.float32),
    )(parts, parts, x, W_root, b_root.reshape(1, -1))


def kernel(x, edge_index, edge_attr, W1, b1, W2, b2, W_root, b_root):
    src = edge_index[0].astype(jnp.int32)
    dst = edge_index[1].astype(jnp.int32)
    x_j = _sc_gather(x, src)
    xj4 = jnp.reshape(x_j, (N_ROWS, LANES))
    ea4 = jnp.reshape(edge_attr, (N_ROWS, PACK * BOND_FDIM))
    msg4 = _tc_messages(ea4, xj4, W1, b1, W2, b2)
    msg = jnp.reshape(msg4, (N_EDGES, DIM_OUT))
    parts = _sc_scatter(msg, dst)
    return _tc_finalize(parts, x, W_root, b_root)
